# Initial kernel scaffold; baseline (speedup 1.0000x reference)
import jax
import jax.numpy as jnp
from jax.experimental import pallas as pl


def kernel(x, weights):
    raise NotImplementedError("write your pallas kernel here")


# speedup vs baseline: 1.4690x; 1.4690x over previous
import functools

import jax
import jax.numpy as jnp
from jax import lax
from jax.experimental import pallas as pl
from jax.experimental.pallas import tpu as pltpu
from jax.experimental.pallas import tpu_sc as plsc

PAD = 1
BATCH = 4
SEQ = 4096
DIM = 512
TABLE_ROWS = 16384

NUM_CORES = 2
NUM_SUBCORES = 16
NW = NUM_CORES * NUM_SUBCORES
ROWS_PER_W = (BATCH * SEQ) // NW
WORKERS_PER_ROW = SEQ // ROWS_PER_W
VREG = 16
VREGS_PER_W = ROWS_PER_W // VREG
CHUNK = 64
NCHUNK = ROWS_PER_W // CHUNK


def _body(x_hbm, w_hbm, out_hbm, xrow, idx, buf0, buf1, sem0, sem1):
    cid = lax.axis_index("c")
    sid = lax.axis_index("s")
    wid = sid * NUM_CORES + cid
    j = wid % WORKERS_PER_ROW
    b = wid // WORKERS_PER_ROW

    pltpu.sync_copy(x_hbm.at[pl.ds(b * SEQ, SEQ)], xrow)

    def _pre(i, acc):
        v = xrow[pl.ds(i * VREG, VREG)]
        return acc + jnp.sum((v != PAD).astype(jnp.int32))

    prefix = lax.fori_loop(0, j * VREGS_PER_W, _pre, jnp.int32(0))

    base_vreg = j * VREGS_PER_W

    def _pos(k, pfx):
        v = xrow[pl.ds((base_vreg + k) * VREG, VREG)]
        m = (v != PAD).astype(jnp.int32)
        c = plsc.cumsum(m)
        idx[pl.ds(k * VREG, VREG)] = (pfx + c) * m + PAD
        return pfx + jnp.sum(m)

    lax.fori_loop(0, VREGS_PER_W, _pos, prefix)

    out_base = wid * ROWS_PER_W
    bufs = (buf0, buf1)
    sems = (sem0, sem1)
    copies = [None] * NCHUNK

    def _start(c):
        return pltpu.async_copy(
            w_hbm.at[idx.at[pl.ds(c * CHUNK, CHUNK)]], bufs[c % 2], sems[c % 2]
        )

    copies[0] = _start(0)
    for c in range(NCHUNK):
        copies[c].wait()
        if c + 1 < NCHUNK:
            copies[c + 1] = _start(c + 1)
        pltpu.sync_copy(bufs[c % 2], out_hbm.at[pl.ds(out_base + c * CHUNK, CHUNK)])


@functools.partial(
    pl.kernel,
    mesh=plsc.VectorSubcoreMesh(core_axis_name="c", subcore_axis_name="s"),
    out_type=jax.ShapeDtypeStruct((BATCH * SEQ, DIM), jnp.float32),
    compiler_params=pltpu.CompilerParams(needs_layout_passes=False),
    scratch_types=[
        pltpu.VMEM((SEQ,), jnp.int32),
        pltpu.VMEM((ROWS_PER_W,), jnp.int32),
        pltpu.VMEM((CHUNK, DIM), jnp.float32),
        pltpu.VMEM((CHUNK, DIM), jnp.float32),
        pltpu.SemaphoreType.DMA,
        pltpu.SemaphoreType.DMA,
    ],
)
def _pos_lookup(x_hbm, w_hbm, out_hbm, xrow, idx, buf0, buf1, sem0, sem1):
    _body(x_hbm, w_hbm, out_hbm, xrow, idx, buf0, buf1, sem0, sem1)


def kernel(x, weights):
    out = _pos_lookup(x.reshape(-1), weights)
    return lax.stop_gradient(out.reshape(BATCH, SEQ, DIM))


# speedup vs baseline: 1.5709x; 1.0693x over previous
import functools

import jax
import jax.numpy as jnp
from jax import lax
from jax.experimental import pallas as pl
from jax.experimental.pallas import tpu as pltpu
from jax.experimental.pallas import tpu_sc as plsc

PAD = 1
BATCH = 4
SEQ = 4096
DIM = 512
TABLE_ROWS = 16384

NUM_CORES = 2
NUM_SUBCORES = 16
NW = NUM_CORES * NUM_SUBCORES
TOK_PER_W = (BATCH * SEQ) // NW
WORKERS_PER_ROW = SEQ // TOK_PER_W
VREG = 16
VREGS_PER_W = TOK_PER_W // VREG
CHUNK = 64
NCHUNK = TOK_PER_W // CHUNK
NBUF = 3


def _body(x_hbm, w_hbm, out_hbm, xchunk, idx, cnt_v, counts_v,
          bufs, gsems, wsems, counts_sh):
    cid = lax.axis_index("c")
    sid = lax.axis_index("s")
    wid = cid * NUM_SUBCORES + sid
    j = sid % WORKERS_PER_ROW
    lrb = sid - j

    pltpu.sync_copy(x_hbm.at[pl.ds(wid * TOK_PER_W, TOK_PER_W)], xchunk)

    acc = jnp.zeros((VREG,), jnp.int32)
    for k in range(VREGS_PER_W):
        v = xchunk[pl.ds(k * VREG, VREG)]
        acc = acc + plsc.all_reduce_population_count(v != PAD)
    cnt_v[...] = acc

    pltpu.sync_copy(cnt_v, counts_sh.at[pl.ds(sid * VREG, VREG)])
    plsc.subcore_barrier()
    pltpu.sync_copy(counts_sh, counts_v)
    pfx = jnp.zeros((VREG,), jnp.int32)
    for k in range(WORKERS_PER_ROW):
        ck = counts_v[pl.ds((lrb + k) * VREG, VREG)]
        pfx = pfx + ck * (j > k).astype(jnp.int32)

    for k in range(VREGS_PER_W):
        v = xchunk[pl.ds(k * VREG, VREG)]
        mb = v != PAD
        m = mb.astype(jnp.int32)
        c = plsc.cumsum(m)
        idx[pl.ds(k * VREG, VREG)] = (pfx + c) * m + PAD
        pfx = pfx + plsc.all_reduce_population_count(mb)

    out_base = wid * TOK_PER_W
    gcopies = [None] * NCHUNK
    wcopies = [None] * NCHUNK

    def _gather(c):
        return pltpu.async_copy(
            w_hbm.at[idx.at[pl.ds(c * CHUNK, CHUNK)]], bufs[c % NBUF],
            gsems[c % NBUF],
        )

    def _write(c):
        return pltpu.async_copy(
            bufs[c % NBUF], out_hbm.at[pl.ds(out_base + c * CHUNK, CHUNK)],
            wsems[c % NBUF],
        )

    gcopies[0] = _gather(0)
    gcopies[1] = _gather(1)
    for c in range(NCHUNK):
        gcopies[c].wait()
        if c + 2 < NCHUNK:
            if c >= 1:
                wcopies[c - 1].wait()
            gcopies[c + 2] = _gather(c + 2)
        wcopies[c] = _write(c)
    for c in range(NCHUNK - NBUF, NCHUNK):
        wcopies[c].wait()


@functools.partial(
    pl.kernel,
    mesh=plsc.VectorSubcoreMesh(core_axis_name="c", subcore_axis_name="s"),
    out_type=jax.ShapeDtypeStruct((BATCH * SEQ, DIM), jnp.float32),
    compiler_params=pltpu.CompilerParams(needs_layout_passes=False),
    scratch_types=[
        pltpu.VMEM((TOK_PER_W,), jnp.int32),
        pltpu.VMEM((TOK_PER_W,), jnp.int32),
        pltpu.VMEM((VREG,), jnp.int32),
        pltpu.VMEM((NUM_SUBCORES * VREG,), jnp.int32),
        [pltpu.VMEM((CHUNK, DIM), jnp.float32) for _ in range(NBUF)],
        [pltpu.SemaphoreType.DMA for _ in range(NBUF)],
        [pltpu.SemaphoreType.DMA for _ in range(NBUF)],
        pltpu.VMEM_SHARED((NUM_SUBCORES * VREG,), jnp.int32),
    ],
)
def _pos_lookup(x_hbm, w_hbm, out_hbm, xchunk, idx, cnt_v, counts_v,
                bufs, gsems, wsems, counts_sh):
    _body(x_hbm, w_hbm, out_hbm, xchunk, idx, cnt_v, counts_v,
          bufs, gsems, wsems, counts_sh)


def kernel(x, weights):
    out = _pos_lookup(x.reshape(-1), weights)
    return lax.stop_gradient(out.reshape(BATCH, SEQ, DIM))
